# Initial kernel scaffold; baseline (speedup 1.0000x reference)
import jax
import jax.numpy as jnp
from jax.experimental import pallas as pl


def kernel(x, adj, W0a, b0a, g0a, be0a, W1a, b1a, g1a, be1a, wpool1, W0b, b0b, g0b, be0b, W1b, b1b, g1b, be1b, wpool2, lin1_W, lin1_b, lin2_W, lin2_b):
    raise NotImplementedError("write your pallas kernel here")


# speedup vs baseline: 2.6455x; 2.6455x over previous
import functools
import math

import jax
import jax.numpy as jnp
from jax import lax
from jax.experimental import pallas as pl

B, N1, C0 = 16, 512, 512
K1 = 128
K2 = 32
H = 64
EPS = 1e-5
F32 = jnp.float32
HI = lax.Precision.HIGHEST


def _mm(a, b):
    return jnp.dot(a, b, preferred_element_type=F32, precision=HI)


def _rb(a):
    return a.astype(jnp.bfloat16).astype(F32)


def _mmb(a, b):
    return jnp.dot(_rb(a), _rb(b), preferred_element_type=F32)


def _dgb(a, b, dims):
    return lax.dot_general(_rb(a), _rb(b), dims, preferred_element_type=F32)


def _dg(a, b, dims):
    return lax.dot_general(a, b, dims, preferred_element_type=F32,
                           precision=HI)


def _norm_cols(a2):
    rowsum = jnp.sum(a2, axis=1, keepdims=True)
    return lax.rsqrt(jnp.maximum(rowsum, 1.0))


def _diag_one(a):
    n = a.shape[0]
    r = lax.broadcasted_iota(jnp.int32, (n, n), 0)
    c = lax.broadcasted_iota(jnp.int32, (n, n), 1)
    return jnp.where(r == c, 1.0, a)


def _gcn(a, x, W, b_row):
    a2 = _diag_one(a)
    deg = _norm_cols(a2)
    adj_n = (deg * a2) * jnp.transpose(deg)
    t = _mmb(x, W)
    return jax.nn.relu(_mmb(adj_n, t) + b_row)


def _stats(y):
    s = jnp.sum(y, axis=1, keepdims=True)
    ss = jnp.sum(y * y, axis=1, keepdims=True)
    return jnp.concatenate([s, ss], axis=1)


def _accum_stats(st_ref, y):
    @pl.when(pl.program_id(0) == 0)
    def _():
        st_ref[...] = jnp.zeros_like(st_ref)
    st_ref[...] = st_ref[...] + _stats(y)


def _bn_apply(y, st, count, g_col, be_col):
    m = st[:, 0:1] / count
    v = st[:, 1:2] / count - m * m
    return (y - m) * lax.rsqrt(v + EPS) * g_col + be_col


def _t_exact(m, n):
    del n
    return jnp.transpose(m)


def _rank_row(s_col, s_row):
    n = s_col.shape[0]
    ri = lax.broadcasted_iota(jnp.int32, (n, n), 0)
    ci = lax.broadcasted_iota(jnp.int32, (n, n), 1)
    cmp = (s_col > s_row) | ((s_col == s_row) & (ri < ci))
    return jnp.sum(jnp.where(cmp, 1.0, 0.0), axis=0, keepdims=True)


def _onehot(rank_row, k):
    n = rank_row.shape[1]
    rows = lax.broadcasted_iota(jnp.int32, (k, n), 0).astype(F32)
    return jnp.where(rows == rank_row, 1.0, 0.0)


def _c1_body(x_ref, adj_ref, W_ref, b_ref, y_ref, st_ref):
    y = _gcn(adj_ref[0], x_ref[0], W_ref[...], b_ref[...])
    y_ref[0] = y
    _accum_stats(st_ref, y)


def _c2_body(y1_ref, st1_ref, adj_ref, g_ref, be_ref, W_ref, b_ref,
             y_ref, st_ref):
    y1n = _bn_apply(y1_ref[0], st1_ref[...], float(B * H),
                    g_ref[...], be_ref[...])
    y = _gcn(adj_ref[0], y1n, W_ref[...], b_ref[...])
    y_ref[0] = y
    _accum_stats(st_ref, y)


def _c3_body(y2_ref, st2_ref, adj_ref, g_ref, be_ref, w_ref,
             W0b_ref, b0b_ref, y3_ref, ap_ref, st_ref):
    x2 = jax.nn.relu(_bn_apply(y2_ref[0], st2_ref[...], float(B * N1),
                               g_ref[...], be_ref[...]))
    w = w_ref[...]
    nrm = jnp.sqrt(jnp.sum(w * w))
    s_col = jnp.tanh(_dgb(x2, w, (((1,), (1,)), ((), ()))) / nrm)
    s_row = _t_exact(s_col, N1)
    P = _onehot(_rank_row(s_col, s_row), K1)
    sk = jnp.sum(P * s_row, axis=1, keepdims=True)
    xp = _mm(P, x2) * sk
    a = adj_ref[0]
    Pa = _mm(P, a)
    Bm = _dg(Pa, P, (((1,), (1,)), ((), ())))
    BmT = _dg(Bm, jnp.eye(K1, dtype=F32), (((0,), (0,)), ((), ())))
    ap = jnp.minimum(Bm + BmT, 1.0)
    ap_ref[0] = ap
    y = _gcn(ap, xp, W0b_ref[...], b0b_ref[...])
    y3_ref[0] = y
    _accum_stats(st_ref, y)


def _c4_body(y3_ref, st3_ref, ap_ref, g_ref, be_ref, W_ref, b_ref,
             y_ref, st_ref):
    y3n = _bn_apply(y3_ref[0], st3_ref[...], float(B * H),
                    g_ref[...], be_ref[...])
    y = _gcn(ap_ref[0], y3n, W_ref[...], b_ref[...])
    y_ref[0] = y
    _accum_stats(st_ref, y)


def _c5_body(y4_ref, st4_ref, g_ref, be_ref, w_ref,
             l1W_ref, l1b_ref, l2W_ref, l2b_ref, out_ref):
    x4 = jax.nn.relu(_bn_apply(y4_ref[0], st4_ref[...], float(B * K1),
                               g_ref[...], be_ref[...]))
    w = w_ref[...]
    nrm = jnp.sqrt(jnp.sum(w * w))
    s_col = jnp.tanh(_dgb(x4, w, (((1,), (1,)), ((), ()))) / nrm)
    s_row = _t_exact(s_col, K1)
    P2 = _onehot(_rank_row(s_col, s_row), K2)
    sk = jnp.sum(P2 * s_row, axis=1, keepdims=True)
    xp2 = _mm(P2, x4) * sk
    u = jnp.sum(_rb(xp2) * _rb(l1W_ref[...]), axis=1,
                keepdims=True)
    z = jax.nn.relu(u + l1b_ref[0, 0])
    o = jnp.sum(_rb(z) * _rb(l2W_ref[...]), axis=0, keepdims=True)
    out_ref[0] = jax.nn.relu(o + l2b_ref[...])


def _bspec(shape, batched=True):
    if batched:
        nd = len(shape) - 1
        return pl.BlockSpec((1,) + shape[1:],
                            lambda b: (b,) + (0,) * nd)
    return pl.BlockSpec(shape, lambda b: (0,) * len(shape))


def _call(body, grid, ins, outs):
    in_specs = [_bspec(s, bt) for s, bt in ins]
    out_specs = [_bspec(s, bt) for s, bt in outs]
    out_shape = [jax.ShapeDtypeStruct(s, F32) for s, _ in outs]
    return pl.pallas_call(
        body, grid=(grid,), in_specs=in_specs, out_specs=out_specs,
        out_shape=out_shape)


def kernel(x, adj, W0a, b0a, g0a, be0a, W1a, b1a, g1a, be1a, wpool1,
           W0b, b0b, g0b, be0b, W1b, b1b, g1b, be1b, wpool2,
           lin1_W, lin1_b, lin2_W, lin2_b):
    col = lambda v: v.reshape(-1, 1)
    row = lambda v: v.reshape(1, -1)

    y1, st1 = _call(_c1_body, B,
                    [((B, N1, C0), True), ((B, N1, N1), True),
                     ((C0, H), False), ((1, H), False)],
                    [((B, N1, H), True), ((N1, 2), False)])(
        x, adj, W0a, row(b0a))

    y2, st2 = _call(_c2_body, B,
                    [((B, N1, H), True), ((N1, 2), False), ((B, N1, N1), True),
                     ((N1, 1), False), ((N1, 1), False),
                     ((H, C0), False), ((1, C0), False)],
                    [((B, N1, C0), True), ((N1, 2), False)])(
        y1, st1, adj, col(g0a), col(be0a), W1a, row(b1a))

    y3, ap, st3 = _call(_c3_body, B,
                        [((B, N1, C0), True), ((N1, 2), False),
                         ((B, N1, N1), True),
                         ((N1, 1), False), ((N1, 1), False), ((1, C0), False),
                         ((C0, H), False), ((1, H), False)],
                        [((B, K1, H), True), ((B, K1, K1), True),
                         ((K1, 2), False)])(
        y2, st2, adj, col(g1a), col(be1a), row(wpool1), W0b, row(b0b))

    y4, st4 = _call(_c4_body, B,
                    [((B, K1, H), True), ((K1, 2), False),
                     ((B, K1, K1), True),
                     ((K1, 1), False), ((K1, 1), False),
                     ((H, K1), False), ((1, K1), False)],
                    [((B, K1, K1), True), ((K1, 2), False)])(
        y3, st3, ap, col(g0b), col(be0b), W1b, row(b1b))

    out = _call(_c5_body, B,
                [((B, K1, K1), True), ((K1, 2), False),
                 ((K1, 1), False), ((K1, 1), False), ((1, K1), False),
                 ((1, K1), False), ((1, 1), False),
                 ((K2, 10), False), ((1, 10), False)],
                [((B, 1, 10), True)])(
        y4, st4, col(g1b), col(be1b), row(wpool2),
        lin1_W.reshape(1, K1), lin1_b.reshape(1, 1), lin2_W,
        lin2_b.reshape(1, 10))[0]

    return (out, jnp.float32(0.0))
